# baseline (speedup 1.0000x reference)
import functools

import jax
import jax.numpy as jnp
from jax import lax
from jax.experimental import pallas as pl
from jax.experimental.pallas import tpu as pltpu
from jax.experimental.pallas import tpu_sc as plsc

B, S, HID = 1, 2048, 1024
E, TOPK, FF = 8, 2, 512
NORM_MIN = 1e-12
T = B * S

BTF = 256
NSP = T * TOPK + E * BTF
NB = NSP // BTF
CBT = 512

NC, NS = 2, 16
NW = NC * NS
TPW = T // NW
CH = 32
C2 = 16
NCH = TPW // C2

_HIGH = jax.lax.Precision.HIGHEST


def _router_body(x_ref, gwt_ref, bias_ref, logits_ref, dst1_ref, dst2_ref,
                 w1_ref, w2_ref, blk_e_ref, nact_ref):
    logits = jnp.dot(x_ref[...], gwt_ref[...],
                     preferred_element_type=jnp.float32)
    m = jnp.max(logits, axis=1, keepdims=True)
    p = jnp.exp(logits - m)
    sm = p / jnp.sum(p, axis=1, keepdims=True)
    corrected = sm + bias_ref[...]
    col = jax.lax.broadcasted_iota(jnp.int32, (T, E), 1)
    m1 = jnp.max(corrected, axis=1, keepdims=True)
    a1 = jnp.min(jnp.where(corrected == m1, col, E), axis=1, keepdims=True)
    oh1 = col == a1
    c2 = jnp.where(oh1, -jnp.inf, corrected)
    m2 = jnp.max(c2, axis=1, keepdims=True)
    a2 = jnp.min(jnp.where(c2 == m2, col, E), axis=1, keepdims=True)
    oh2 = col == a2
    w1 = jnp.sum(jnp.where(oh1, sm, 0.0), axis=1, keepdims=True)
    w2 = jnp.sum(jnp.where(oh2, sm, 0.0), axis=1, keepdims=True)
    s = jnp.maximum(w1 + w2, NORM_MIN)

    sel = oh1.astype(jnp.float32) + oh2.astype(jnp.float32)
    run = jnp.zeros((1, E), dtype=jnp.float32)
    parts = []
    r_io = jax.lax.broadcasted_iota(jnp.int32, (CBT, CBT), 0)
    c_io = jax.lax.broadcasted_iota(jnp.int32, (CBT, CBT), 1)
    tril = (r_io > c_io).astype(jnp.float32)
    for i in range(T // CBT):
        blk = sel[i * CBT:(i + 1) * CBT, :]
        local = jnp.dot(tril, blk, preferred_element_type=jnp.float32,
                        precision=_HIGH)
        parts.append(local + run)
        run = run + jnp.sum(blk, axis=0, keepdims=True)
    cex = jnp.concatenate(parts, axis=0)
    counts = run

    blocks = jnp.floor((counts + (BTF - 1)) * (1.0 / BTF))
    e_r = jax.lax.broadcasted_iota(jnp.int32, (E, E), 0)
    e_c = jax.lax.broadcasted_iota(jnp.int32, (E, E), 1)
    upper_incl = (e_r <= e_c).astype(jnp.float32)
    cumb = jnp.dot(blocks, upper_incl, preferred_element_type=jnp.float32,
                   precision=_HIGH)
    off = (cumb - blocks) * float(BTF)

    pos = off + cex
    dst1 = jnp.sum(jnp.where(oh1, pos, 0.0), axis=1, keepdims=True)
    dst2 = jnp.sum(jnp.where(oh2, pos, 0.0), axis=1, keepdims=True)

    logits_ref[...] = logits
    dst1_ref[...] = dst1.astype(jnp.int32)
    dst2_ref[...] = dst2.astype(jnp.int32)
    w1_ref[...] = w1 / s
    w2_ref[...] = w2 / s

    b_io = jax.lax.broadcasted_iota(jnp.int32, (NB, E), 0).astype(jnp.float32)
    cumb_b = jnp.broadcast_to(cumb, (NB, E))
    blk_e_ref[...] = jnp.sum((b_io >= cumb_b).astype(jnp.int32), axis=1,
                             keepdims=True)
    nact_ref[...] = cumb[:, E - 1:E].astype(jnp.int32)


def _dispatch_body(x_hbm, d1_hbm, d2_hbm, xs_hbm,
                   x_v, idx1_v, idx2_v, sem1, sem2):
    wid = lax.axis_index("s") * NC + lax.axis_index("c")
    for ch in range(TPW // CH):
        base = wid * TPW + ch * CH
        pltpu.sync_copy(x_hbm.at[pl.ds(base, CH)], x_v)
        pltpu.sync_copy(d1_hbm.at[pl.ds(base, CH)], idx1_v)
        pltpu.sync_copy(d2_hbm.at[pl.ds(base, CH)], idx2_v)
        c1 = pltpu.async_copy(x_v, xs_hbm.at[idx1_v], sem1)
        c2 = pltpu.async_copy(x_v, xs_hbm.at[idx2_v], sem2)
        c1.wait()
        c2.wait()


def _ffn_body(blk_e_ref, nact_ref, xs_ref, gp_ref, up_ref, dp_ref, ys_ref):
    b = pl.program_id(0)

    @pl.when(b < nact_ref[0])
    def _():
        x = xs_ref[...]
        g = jnp.dot(x, gp_ref[0], preferred_element_type=jnp.float32)
        u = jnp.dot(x, up_ref[0], preferred_element_type=jnp.float32)
        h = (g * jax.nn.sigmoid(g)) * u
        ys_ref[...] = jnp.dot(h, dp_ref[0],
                              preferred_element_type=jnp.float32)


def _combine_body(ys_hbm, d1_hbm, d2_hbm, w1_hbm, w2_hbm, out_hbm,
                  idx1_v, idx2_v, w1_v, w2_v,
                  r1a, r2a, oa, r1b, r2b, ob,
                  sga, sgb, swa, swb):
    wid = lax.axis_index("s") * NC + lax.axis_index("c")
    base = wid * TPW
    pltpu.sync_copy(d1_hbm.at[pl.ds(base, TPW)], idx1_v)
    pltpu.sync_copy(d2_hbm.at[pl.ds(base, TPW)], idx2_v)
    pltpu.sync_copy(w1_hbm.at[pl.ds(base, TPW)], w1_v)
    pltpu.sync_copy(w2_hbm.at[pl.ds(base, TPW)], w2_v)

    bufs = [(r1a, r2a, oa, sga, swa), (r1b, r2b, ob, sgb, swb)]

    def gathers(ch, r1, r2, sg):
        i1 = idx1_v.at[pl.ds(ch * C2, C2)]
        i2 = idx2_v.at[pl.ds(ch * C2, C2)]
        return (pltpu.async_copy(ys_hbm.at[i1], r1, sg),
                pltpu.async_copy(ys_hbm.at[i2], r2, sg))

    pending_g = {0: gathers(0, r1a, r2a, sga)}
    pending_w = {}
    for ch in range(NCH):
        r1, r2, o, sg, sw = bufs[ch % 2]
        for c in pending_g.pop(ch):
            c.wait()
        if ch + 1 < NCH:
            n1, n2, _, nsg, _ = bufs[(ch + 1) % 2]
            if ch - 1 in pending_w:
                pending_w.pop(ch - 1).wait()
            pending_g[ch + 1] = gathers(ch + 1, n1, n2, nsg)
        for j in range(C2):
            jj = jnp.full((16,), j, dtype=jnp.int32)
            w1s = w1_v[pl.ds(ch * C2, 16)]
            w2s = w2_v[pl.ds(ch * C2, 16)]
            w1b = w1s.at[jj].get(mode="promise_in_bounds")
            w2b = w2s.at[jj].get(mode="promise_in_bounds")

            def body(ci, carry, j=j, w1b=w1b, w2b=w2b, r1=r1, r2=r2, o=o):
                for u in range(4):
                    sl = pl.ds(ci * 64 + u * 16, 16)
                    o[j, sl] = w1b * r1[j, sl] + w2b * r2[j, sl]
                return carry

            lax.fori_loop(0, HID // 64, body, 0)
        pending_w[ch] = pltpu.async_copy(
            o, out_hbm.at[pl.ds(base + ch * C2, C2)], sw)
    for c in pending_w.values():
        c.wait()


def kernel(hidden_states, gate_w, e_score_correction_bias, gate_proj,
           up_proj, down_proj):
    x = hidden_states.reshape(T, HID).astype(jnp.float32)

    logits, dst1, dst2, w1, w2, blk_e, nact = pl.pallas_call(
        _router_body,
        out_shape=[
            jax.ShapeDtypeStruct((T, E), jnp.float32),
            jax.ShapeDtypeStruct((T, 1), jnp.int32),
            jax.ShapeDtypeStruct((T, 1), jnp.int32),
            jax.ShapeDtypeStruct((T, 1), jnp.float32),
            jax.ShapeDtypeStruct((T, 1), jnp.float32),
            jax.ShapeDtypeStruct((NB, 1), jnp.int32),
            jax.ShapeDtypeStruct((1, 1), jnp.int32),
        ],
    )(x, gate_w.T, e_score_correction_bias)

    d1 = dst1.reshape(T)
    d2 = dst2.reshape(T)
    w1f = w1.reshape(T)
    w2f = w2.reshape(T)

    mesh = plsc.VectorSubcoreMesh(core_axis_name="c", subcore_axis_name="s",
                                  num_cores=NC, num_subcores=NS)

    xs = pl.kernel(
        _dispatch_body,
        out_type=jax.ShapeDtypeStruct((NSP, HID), jnp.float32),
        mesh=mesh,
        scratch_types=[
            pltpu.VMEM((CH, HID), jnp.float32),
            pltpu.VMEM((CH,), jnp.int32),
            pltpu.VMEM((CH,), jnp.int32),
            pltpu.SemaphoreType.DMA,
            pltpu.SemaphoreType.DMA,
        ],
    )(x, d1, d2)

    ys = pl.pallas_call(
        _ffn_body,
        grid_spec=pltpu.PrefetchScalarGridSpec(
            num_scalar_prefetch=2,
            grid=(NB,),
            in_specs=[
                pl.BlockSpec((BTF, HID),
                             lambda b, be, na: (jnp.minimum(b, na[0] - 1), 0)),
                pl.BlockSpec((1, HID, FF),
                             lambda b, be, na:
                             (be[jnp.minimum(b, na[0] - 1)], 0, 0)),
                pl.BlockSpec((1, HID, FF),
                             lambda b, be, na:
                             (be[jnp.minimum(b, na[0] - 1)], 0, 0)),
                pl.BlockSpec((1, FF, HID),
                             lambda b, be, na:
                             (be[jnp.minimum(b, na[0] - 1)], 0, 0)),
            ],
            out_specs=pl.BlockSpec(
                (BTF, HID), lambda b, be, na: (jnp.minimum(b, na[0] - 1), 0)),
        ),
        out_shape=jax.ShapeDtypeStruct((NSP, HID), jnp.float32),
    )(blk_e.reshape(NB), nact.reshape(1), xs, gate_proj, up_proj, down_proj)

    final = pl.kernel(
        _combine_body,
        out_type=jax.ShapeDtypeStruct((T, HID), jnp.float32),
        mesh=mesh,
        scratch_types=[
            pltpu.VMEM((TPW,), jnp.int32),
            pltpu.VMEM((TPW,), jnp.int32),
            pltpu.VMEM((TPW,), jnp.float32),
            pltpu.VMEM((TPW,), jnp.float32),
            pltpu.VMEM((C2, HID), jnp.float32),
            pltpu.VMEM((C2, HID), jnp.float32),
            pltpu.VMEM((C2, HID), jnp.float32),
            pltpu.VMEM((C2, HID), jnp.float32),
            pltpu.VMEM((C2, HID), jnp.float32),
            pltpu.VMEM((C2, HID), jnp.float32),
            pltpu.SemaphoreType.DMA,
            pltpu.SemaphoreType.DMA,
            pltpu.SemaphoreType.DMA,
            pltpu.SemaphoreType.DMA,
        ],
    )(ys, d1, d2, w1f, w2f)

    return (final.reshape(-1), logits.reshape(-1))


# speedup vs baseline: 1.0098x; 1.0098x over previous
import functools

import jax
import jax.numpy as jnp
from jax import lax
from jax.experimental import pallas as pl
from jax.experimental.pallas import tpu as pltpu
from jax.experimental.pallas import tpu_sc as plsc

B, S, HID = 1, 2048, 1024
E, TOPK, FF = 8, 2, 512
NORM_MIN = 1e-12
T = B * S

BTF = 256
NSP = T * TOPK + E * BTF
NB = NSP // BTF
CBT = 512

NC, NS = 2, 16
NW = NC * NS
TPW = T // NW
CH = 64
C2 = 16
NCH = TPW // C2

_HIGH = jax.lax.Precision.HIGHEST


def _router_body(logits_in_ref, bias_ref, logits_ref, dst1_ref, dst2_ref,
                 w1_ref, w2_ref, blk_e_ref, nact_ref):
    logits = logits_in_ref[...]
    m = jnp.max(logits, axis=1, keepdims=True)
    p = jnp.exp(logits - m)
    sm = p / jnp.sum(p, axis=1, keepdims=True)
    corrected = sm + bias_ref[...]
    col = jax.lax.broadcasted_iota(jnp.int32, (T, E), 1)
    m1 = jnp.max(corrected, axis=1, keepdims=True)
    a1 = jnp.min(jnp.where(corrected == m1, col, E), axis=1, keepdims=True)
    oh1 = col == a1
    c2 = jnp.where(oh1, -jnp.inf, corrected)
    m2 = jnp.max(c2, axis=1, keepdims=True)
    a2 = jnp.min(jnp.where(c2 == m2, col, E), axis=1, keepdims=True)
    oh2 = col == a2
    w1 = jnp.sum(jnp.where(oh1, sm, 0.0), axis=1, keepdims=True)
    w2 = jnp.sum(jnp.where(oh2, sm, 0.0), axis=1, keepdims=True)
    s = jnp.maximum(w1 + w2, NORM_MIN)

    sel = oh1.astype(jnp.float32) + oh2.astype(jnp.float32)
    run = jnp.zeros((1, E), dtype=jnp.float32)
    parts = []
    r_io = jax.lax.broadcasted_iota(jnp.int32, (CBT, CBT), 0)
    c_io = jax.lax.broadcasted_iota(jnp.int32, (CBT, CBT), 1)
    tril = (r_io > c_io).astype(jnp.float32)
    for i in range(T // CBT):
        blk = sel[i * CBT:(i + 1) * CBT, :]
        local = jnp.dot(tril, blk, preferred_element_type=jnp.float32,
                        precision=_HIGH)
        parts.append(local + run)
        run = run + jnp.sum(blk, axis=0, keepdims=True)
    cex = jnp.concatenate(parts, axis=0)
    counts = run

    blocks = jnp.floor((counts + (BTF - 1)) * (1.0 / BTF))
    e_r = jax.lax.broadcasted_iota(jnp.int32, (E, E), 0)
    e_c = jax.lax.broadcasted_iota(jnp.int32, (E, E), 1)
    upper_incl = (e_r <= e_c).astype(jnp.float32)
    cumb = jnp.dot(blocks, upper_incl, preferred_element_type=jnp.float32,
                   precision=_HIGH)
    off = (cumb - blocks) * float(BTF)

    pos = off + cex
    dst1 = jnp.sum(jnp.where(oh1, pos, 0.0), axis=1, keepdims=True)
    dst2 = jnp.sum(jnp.where(oh2, pos, 0.0), axis=1, keepdims=True)

    logits_ref[...] = logits
    dst1_ref[...] = dst1.astype(jnp.int32)
    dst2_ref[...] = dst2.astype(jnp.int32)
    w1_ref[...] = w1 / s
    w2_ref[...] = w2 / s

    b_io = jax.lax.broadcasted_iota(jnp.int32, (NB, E), 0).astype(jnp.float32)
    cumb_b = jnp.broadcast_to(cumb, (NB, E))
    blk_e_ref[...] = jnp.sum((b_io >= cumb_b).astype(jnp.int32), axis=1,
                             keepdims=True)
    nact_ref[...] = cumb[:, E - 1:E].astype(jnp.int32)


def _dispatch_body(x_hbm, d1_hbm, d2_hbm, xs_hbm,
                   x_v, idx1_v, idx2_v, sem1, sem2):
    wid = lax.axis_index("s") * NC + lax.axis_index("c")
    for ch in range(TPW // CH):
        base = wid * TPW + ch * CH
        pltpu.sync_copy(x_hbm.at[pl.ds(base, CH)], x_v)
        pltpu.sync_copy(d1_hbm.at[pl.ds(base, CH)], idx1_v)
        pltpu.sync_copy(d2_hbm.at[pl.ds(base, CH)], idx2_v)
        c1 = pltpu.async_copy(x_v, xs_hbm.at[idx1_v], sem1)
        c2 = pltpu.async_copy(x_v, xs_hbm.at[idx2_v], sem2)
        c1.wait()
        c2.wait()


def _ffn_body(blk_e_ref, nact_ref, xs_ref, gp_ref, up_ref, dp_ref, ys_ref):
    b = pl.program_id(0)

    @pl.when(b < nact_ref[0])
    def _():
        x = xs_ref[...]
        g = jnp.dot(x, gp_ref[0], preferred_element_type=jnp.float32)
        u = jnp.dot(x, up_ref[0], preferred_element_type=jnp.float32)
        h = (g * jax.nn.sigmoid(g)) * u
        ys_ref[...] = jnp.dot(h, dp_ref[0],
                              preferred_element_type=jnp.float32)


def _combine_body(ys_hbm, d1_hbm, d2_hbm, w1_hbm, w2_hbm, out_hbm,
                  idx1_v, idx2_v, w1_v, w2_v,
                  r1a, r2a, oa, r1b, r2b, ob,
                  sga, sgb, swa, swb):
    wid = lax.axis_index("s") * NC + lax.axis_index("c")
    base = wid * TPW
    pltpu.sync_copy(d1_hbm.at[pl.ds(base, TPW)], idx1_v)
    pltpu.sync_copy(d2_hbm.at[pl.ds(base, TPW)], idx2_v)
    pltpu.sync_copy(w1_hbm.at[pl.ds(base, TPW)], w1_v)
    pltpu.sync_copy(w2_hbm.at[pl.ds(base, TPW)], w2_v)

    bufs = [(r1a, r2a, oa, sga, swa), (r1b, r2b, ob, sgb, swb)]

    def gathers(ch, r1, r2, sg):
        i1 = idx1_v.at[pl.ds(ch * C2, C2)]
        i2 = idx2_v.at[pl.ds(ch * C2, C2)]
        return (pltpu.async_copy(ys_hbm.at[i1], r1, sg),
                pltpu.async_copy(ys_hbm.at[i2], r2, sg))

    pending_g = {0: gathers(0, r1a, r2a, sga)}
    pending_w = {}
    for ch in range(NCH):
        r1, r2, o, sg, sw = bufs[ch % 2]
        for c in pending_g.pop(ch):
            c.wait()
        if ch + 1 < NCH:
            n1, n2, _, nsg, _ = bufs[(ch + 1) % 2]
            if ch - 1 in pending_w:
                pending_w.pop(ch - 1).wait()
            pending_g[ch + 1] = gathers(ch + 1, n1, n2, nsg)
        for j in range(C2):
            jj = jnp.full((16,), j, dtype=jnp.int32)
            w1s = w1_v[pl.ds(ch * C2, 16)]
            w2s = w2_v[pl.ds(ch * C2, 16)]
            w1b = w1s.at[jj].get(mode="promise_in_bounds")
            w2b = w2s.at[jj].get(mode="promise_in_bounds")

            def body(ci, carry, j=j, w1b=w1b, w2b=w2b, r1=r1, r2=r2, o=o):
                for u in range(4):
                    sl = pl.ds(ci * 64 + u * 16, 16)
                    o[j, sl] = w1b * r1[j, sl] + w2b * r2[j, sl]
                return carry

            lax.fori_loop(0, HID // 64, body, 0)
        pending_w[ch] = pltpu.async_copy(
            o, out_hbm.at[pl.ds(base + ch * C2, C2)], sw)
    for c in pending_w.values():
        c.wait()


def kernel(hidden_states, gate_w, e_score_correction_bias, gate_proj,
           up_proj, down_proj):
    x = hidden_states.reshape(T, HID).astype(jnp.float32)
    logits_in = x @ gate_w.T

    logits, dst1, dst2, w1, w2, blk_e, nact = pl.pallas_call(
        _router_body,
        out_shape=[
            jax.ShapeDtypeStruct((T, E), jnp.float32),
            jax.ShapeDtypeStruct((T, 1), jnp.int32),
            jax.ShapeDtypeStruct((T, 1), jnp.int32),
            jax.ShapeDtypeStruct((T, 1), jnp.float32),
            jax.ShapeDtypeStruct((T, 1), jnp.float32),
            jax.ShapeDtypeStruct((NB, 1), jnp.int32),
            jax.ShapeDtypeStruct((1, 1), jnp.int32),
        ],
    )(logits_in, e_score_correction_bias)

    d1 = dst1.reshape(T)
    d2 = dst2.reshape(T)
    w1f = w1.reshape(T)
    w2f = w2.reshape(T)

    mesh = plsc.VectorSubcoreMesh(core_axis_name="c", subcore_axis_name="s",
                                  num_cores=NC, num_subcores=NS)

    xs = pl.kernel(
        _dispatch_body,
        out_type=jax.ShapeDtypeStruct((NSP, HID), jnp.float32),
        mesh=mesh,
        scratch_types=[
            pltpu.VMEM((CH, HID), jnp.float32),
            pltpu.VMEM((CH,), jnp.int32),
            pltpu.VMEM((CH,), jnp.int32),
            pltpu.SemaphoreType.DMA,
            pltpu.SemaphoreType.DMA,
        ],
    )(x, d1, d2)

    ys = pl.pallas_call(
        _ffn_body,
        grid_spec=pltpu.PrefetchScalarGridSpec(
            num_scalar_prefetch=2,
            grid=(NB,),
            in_specs=[
                pl.BlockSpec((BTF, HID),
                             lambda b, be, na: (jnp.minimum(b, na[0] - 1), 0)),
                pl.BlockSpec((1, HID, FF),
                             lambda b, be, na:
                             (be[jnp.minimum(b, na[0] - 1)], 0, 0)),
                pl.BlockSpec((1, HID, FF),
                             lambda b, be, na:
                             (be[jnp.minimum(b, na[0] - 1)], 0, 0)),
                pl.BlockSpec((1, FF, HID),
                             lambda b, be, na:
                             (be[jnp.minimum(b, na[0] - 1)], 0, 0)),
            ],
            out_specs=pl.BlockSpec(
                (BTF, HID), lambda b, be, na: (jnp.minimum(b, na[0] - 1), 0)),
        ),
        out_shape=jax.ShapeDtypeStruct((NSP, HID), jnp.float32),
    )(blk_e.reshape(NB), nact.reshape(1), xs, gate_proj, up_proj, down_proj)

    final = pl.kernel(
        _combine_body,
        out_type=jax.ShapeDtypeStruct((T, HID), jnp.float32),
        mesh=mesh,
        scratch_types=[
            pltpu.VMEM((TPW,), jnp.int32),
            pltpu.VMEM((TPW,), jnp.int32),
            pltpu.VMEM((TPW,), jnp.float32),
            pltpu.VMEM((TPW,), jnp.float32),
            pltpu.VMEM((C2, HID), jnp.float32),
            pltpu.VMEM((C2, HID), jnp.float32),
            pltpu.VMEM((C2, HID), jnp.float32),
            pltpu.VMEM((C2, HID), jnp.float32),
            pltpu.VMEM((C2, HID), jnp.float32),
            pltpu.VMEM((C2, HID), jnp.float32),
            pltpu.SemaphoreType.DMA,
            pltpu.SemaphoreType.DMA,
            pltpu.SemaphoreType.DMA,
            pltpu.SemaphoreType.DMA,
        ],
    )(ys, d1, d2, w1f, w2f)

    return (final.reshape(-1), logits.reshape(-1))


# speedup vs baseline: 1.0176x; 1.0077x over previous
import functools

import jax
import jax.numpy as jnp
from jax import lax
from jax.experimental import pallas as pl
from jax.experimental.pallas import tpu as pltpu
from jax.experimental.pallas import tpu_sc as plsc

B, S, HID = 1, 2048, 1024
E, TOPK, FF = 8, 2, 512
NORM_MIN = 1e-12
T = B * S

BTF = 256
NSP = T * TOPK + E * BTF
NB = NSP // BTF
CBT = 512

NC, NS = 2, 16
NW = NC * NS
TPW = T // NW
CH = 64
C2 = 16
NCH = TPW // C2

_HIGH = jax.lax.Precision.HIGHEST


def _router_body(logits_in_ref, bias_ref, logits_ref, dst1_ref, dst2_ref,
                 w1_ref, w2_ref, blk_e_ref, nact_ref):
    logits = logits_in_ref[...]
    m = jnp.max(logits, axis=1, keepdims=True)
    p = jnp.exp(logits - m)
    sm = p / jnp.sum(p, axis=1, keepdims=True)
    corrected = sm + bias_ref[...]
    col = jax.lax.broadcasted_iota(jnp.int32, (T, E), 1)
    m1 = jnp.max(corrected, axis=1, keepdims=True)
    a1 = jnp.min(jnp.where(corrected == m1, col, E), axis=1, keepdims=True)
    oh1 = col == a1
    c2 = jnp.where(oh1, -jnp.inf, corrected)
    m2 = jnp.max(c2, axis=1, keepdims=True)
    a2 = jnp.min(jnp.where(c2 == m2, col, E), axis=1, keepdims=True)
    oh2 = col == a2
    w1 = jnp.sum(jnp.where(oh1, sm, 0.0), axis=1, keepdims=True)
    w2 = jnp.sum(jnp.where(oh2, sm, 0.0), axis=1, keepdims=True)
    s = jnp.maximum(w1 + w2, NORM_MIN)

    sel = oh1.astype(jnp.float32) + oh2.astype(jnp.float32)
    run = jnp.zeros((1, E), dtype=jnp.float32)
    parts = []
    r_io = jax.lax.broadcasted_iota(jnp.int32, (CBT, CBT), 0)
    c_io = jax.lax.broadcasted_iota(jnp.int32, (CBT, CBT), 1)
    tril = (r_io > c_io).astype(jnp.float32)
    for i in range(T // CBT):
        blk = sel[i * CBT:(i + 1) * CBT, :]
        local = jnp.dot(tril, blk, preferred_element_type=jnp.float32,
                        precision=_HIGH)
        parts.append(local + run)
        run = run + jnp.sum(blk, axis=0, keepdims=True)
    cex = jnp.concatenate(parts, axis=0)
    counts = run

    blocks = jnp.floor((counts + (BTF - 1)) * (1.0 / BTF))
    e_r = jax.lax.broadcasted_iota(jnp.int32, (E, E), 0)
    e_c = jax.lax.broadcasted_iota(jnp.int32, (E, E), 1)
    upper_incl = (e_r <= e_c).astype(jnp.float32)
    cumb = jnp.dot(blocks, upper_incl, preferred_element_type=jnp.float32,
                   precision=_HIGH)
    off = (cumb - blocks) * float(BTF)

    pos = off + cex
    dst1 = jnp.sum(jnp.where(oh1, pos, 0.0), axis=1, keepdims=True)
    dst2 = jnp.sum(jnp.where(oh2, pos, 0.0), axis=1, keepdims=True)

    logits_ref[...] = logits
    dst1_ref[...] = dst1.astype(jnp.int32)
    dst2_ref[...] = dst2.astype(jnp.int32)
    w1_ref[...] = w1 / s
    w2_ref[...] = w2 / s

    b_io = jax.lax.broadcasted_iota(jnp.int32, (NB, E), 0).astype(jnp.float32)
    cumb_b = jnp.broadcast_to(cumb, (NB, E))
    blk_e_ref[...] = jnp.sum((b_io >= cumb_b).astype(jnp.int32), axis=1,
                             keepdims=True)
    nact_ref[...] = cumb[:, E - 1:E].astype(jnp.int32)


def _dispatch_body(x_hbm, d1_hbm, d2_hbm, xs_hbm,
                   x_v, idx1_v, idx2_v, sem1, sem2):
    wid = lax.axis_index("s") * NC + lax.axis_index("c")
    for ch in range(TPW // CH):
        base = wid * TPW + ch * CH
        pltpu.sync_copy(x_hbm.at[pl.ds(base, CH)], x_v)
        pltpu.sync_copy(d1_hbm.at[pl.ds(base, CH)], idx1_v)
        pltpu.sync_copy(d2_hbm.at[pl.ds(base, CH)], idx2_v)
        c1 = pltpu.async_copy(x_v, xs_hbm.at[idx1_v], sem1)
        c2 = pltpu.async_copy(x_v, xs_hbm.at[idx2_v], sem2)
        c1.wait()
        c2.wait()


def _ffn_body(blk_e_ref, nact_ref, xs_ref, gp_ref, up_ref, dp_ref, ys_ref):
    b = pl.program_id(0)

    @pl.when(b < nact_ref[0])
    def _():
        x = xs_ref[...]
        g = jnp.dot(x, gp_ref[0], preferred_element_type=jnp.float32)
        u = jnp.dot(x, up_ref[0], preferred_element_type=jnp.float32)
        h = (g * jax.nn.sigmoid(g)) * u
        ys_ref[...] = jnp.dot(h, dp_ref[0],
                              preferred_element_type=jnp.float32)


def _combine_body(ys_hbm, d1_hbm, d2_hbm, w1_hbm, w2_hbm, out_hbm,
                  idx1_v, idx2_v, w1_v, w2_v,
                  r1a, r2a, oa, r1b, r2b, ob,
                  sga, sgb, swa, swb):
    wid = lax.axis_index("s") * NC + lax.axis_index("c")
    base = wid * TPW
    pltpu.sync_copy(d1_hbm.at[pl.ds(base, TPW)], idx1_v)
    pltpu.sync_copy(d2_hbm.at[pl.ds(base, TPW)], idx2_v)
    pltpu.sync_copy(w1_hbm.at[pl.ds(base, TPW)], w1_v)
    pltpu.sync_copy(w2_hbm.at[pl.ds(base, TPW)], w2_v)

    bufs = [(r1a, r2a, oa, sga, swa), (r1b, r2b, ob, sgb, swb)]

    def gathers(ch, r1, r2, sg):
        i1 = idx1_v.at[pl.ds(ch * C2, C2)]
        i2 = idx2_v.at[pl.ds(ch * C2, C2)]
        return (pltpu.async_copy(ys_hbm.at[i1], r1, sg),
                pltpu.async_copy(ys_hbm.at[i2], r2, sg))

    pending_g = {0: gathers(0, r1a, r2a, sga)}
    pending_w = {}
    for ch in range(NCH):
        r1, r2, o, sg, sw = bufs[ch % 2]
        for c in pending_g.pop(ch):
            c.wait()
        if ch + 1 < NCH:
            n1, n2, _, nsg, _ = bufs[(ch + 1) % 2]
            if ch - 1 in pending_w:
                pending_w.pop(ch - 1).wait()
            pending_g[ch + 1] = gathers(ch + 1, n1, n2, nsg)
        for j in range(C2):
            jj = jnp.full((16,), j, dtype=jnp.int32)
            w1s = w1_v[pl.ds(ch * C2, 16)]
            w2s = w2_v[pl.ds(ch * C2, 16)]
            w1b = w1s.at[jj].get(mode="promise_in_bounds")
            w2b = w2s.at[jj].get(mode="promise_in_bounds")

            def body(ci, carry, j=j, w1b=w1b, w2b=w2b, r1=r1, r2=r2, o=o):
                for u in range(4):
                    sl = pl.ds(ci * 64 + u * 16, 16)
                    o[pl.ds(j * HID + ci * 64 + u * 16, 16)] = (
                        w1b * r1[j, sl] + w2b * r2[j, sl])
                return carry

            lax.fori_loop(0, HID // 64, body, 0)
        pending_w[ch] = pltpu.async_copy(
            o, out_hbm.at[pl.ds((base + ch * C2) * HID, C2 * HID)], sw)
    for c in pending_w.values():
        c.wait()


def kernel(hidden_states, gate_w, e_score_correction_bias, gate_proj,
           up_proj, down_proj):
    x = hidden_states.reshape(T, HID).astype(jnp.float32)
    logits_in = x @ gate_w.T

    logits, dst1, dst2, w1, w2, blk_e, nact = pl.pallas_call(
        _router_body,
        out_shape=[
            jax.ShapeDtypeStruct((T, E), jnp.float32),
            jax.ShapeDtypeStruct((T, 1), jnp.int32),
            jax.ShapeDtypeStruct((T, 1), jnp.int32),
            jax.ShapeDtypeStruct((T, 1), jnp.float32),
            jax.ShapeDtypeStruct((T, 1), jnp.float32),
            jax.ShapeDtypeStruct((NB, 1), jnp.int32),
            jax.ShapeDtypeStruct((1, 1), jnp.int32),
        ],
    )(logits_in, e_score_correction_bias)

    d1 = dst1.reshape(T)
    d2 = dst2.reshape(T)
    w1f = w1.reshape(T)
    w2f = w2.reshape(T)

    mesh = plsc.VectorSubcoreMesh(core_axis_name="c", subcore_axis_name="s",
                                  num_cores=NC, num_subcores=NS)

    xs = pl.kernel(
        _dispatch_body,
        out_type=jax.ShapeDtypeStruct((NSP, HID), jnp.float32),
        mesh=mesh,
        scratch_types=[
            pltpu.VMEM((CH, HID), jnp.float32),
            pltpu.VMEM((CH,), jnp.int32),
            pltpu.VMEM((CH,), jnp.int32),
            pltpu.SemaphoreType.DMA,
            pltpu.SemaphoreType.DMA,
        ],
    )(x, d1, d2)

    ys = pl.pallas_call(
        _ffn_body,
        grid_spec=pltpu.PrefetchScalarGridSpec(
            num_scalar_prefetch=2,
            grid=(NB,),
            in_specs=[
                pl.BlockSpec((BTF, HID),
                             lambda b, be, na: (jnp.minimum(b, na[0] - 1), 0)),
                pl.BlockSpec((1, HID, FF),
                             lambda b, be, na:
                             (be[jnp.minimum(b, na[0] - 1)], 0, 0)),
                pl.BlockSpec((1, HID, FF),
                             lambda b, be, na:
                             (be[jnp.minimum(b, na[0] - 1)], 0, 0)),
                pl.BlockSpec((1, FF, HID),
                             lambda b, be, na:
                             (be[jnp.minimum(b, na[0] - 1)], 0, 0)),
            ],
            out_specs=pl.BlockSpec(
                (BTF, HID), lambda b, be, na: (jnp.minimum(b, na[0] - 1), 0)),
        ),
        out_shape=jax.ShapeDtypeStruct((NSP, HID), jnp.float32),
    )(blk_e.reshape(NB), nact.reshape(1), xs, gate_proj, up_proj, down_proj)

    final = pl.kernel(
        _combine_body,
        out_type=jax.ShapeDtypeStruct((T * HID,), jnp.float32),
        mesh=mesh,
        scratch_types=[
            pltpu.VMEM((TPW,), jnp.int32),
            pltpu.VMEM((TPW,), jnp.int32),
            pltpu.VMEM((TPW,), jnp.float32),
            pltpu.VMEM((TPW,), jnp.float32),
            pltpu.VMEM((C2, HID), jnp.float32),
            pltpu.VMEM((C2, HID), jnp.float32),
            pltpu.VMEM((C2 * HID,), jnp.float32),
            pltpu.VMEM((C2, HID), jnp.float32),
            pltpu.VMEM((C2, HID), jnp.float32),
            pltpu.VMEM((C2 * HID,), jnp.float32),
            pltpu.SemaphoreType.DMA,
            pltpu.SemaphoreType.DMA,
            pltpu.SemaphoreType.DMA,
            pltpu.SemaphoreType.DMA,
        ],
    )(ys, d1, d2, w1f, w2f)

    return (final, logits.reshape(-1))
